# baseline (speedup 1.0000x reference)
import functools

import jax
import jax.numpy as jnp
from jax import lax
from jax.experimental import pallas as pl
from jax.experimental.pallas import tpu as pltpu
from jax.experimental.pallas import tpu_sc as plsc

_NUM_FIELDS = 26
_VOCAB = 100000
_EMBED_DIM = 16
_BATCH = 16384

_NC = 2
_NS = 16
_L = 16
_NW = _NC * _NS

_PLANES = _NUM_FIELDS * _EMBED_DIM
_PLANES_PER_W = _PLANES // _NW
_UNROLL = 16
_N_UNITS = _BATCH // (_UNROLL * _L)


_mesh = plsc.VectorSubcoreMesh(core_axis_name="c", subcore_axis_name="s")


@functools.partial(
    pl.kernel,
    mesh=_mesh,
    out_type=jax.ShapeDtypeStruct((_PLANES, _BATCH), jnp.int32),
    scratch_types=[
        pltpu.VMEM((_VOCAB,), jnp.float32),
        pltpu.VMEM((_BATCH,), jnp.int32),
        pltpu.SemaphoreType.DMA,
    ],
    compiler_params=pltpu.CompilerParams(
        use_tc_tiling_on_sc=True, needs_layout_passes=False
    ),
)
def _lookup_kernel(tab_hbm, idx_hbm, out_hbm, plane_v, buf_v, semp):
    wid = lax.axis_index("s") * _NC + lax.axis_index("c")
    fp0 = wid * _PLANES_PER_W

    def wait_plane(fp):
        pltpu.make_async_copy(tab_hbm.at[fp], plane_v, semp).wait()

    def gather_inplace(_, carry):
        j = carry
        base = j * (_UNROLL * _L)
        for u in range(_UNROLL):
            sl = pl.ds(base + u * _L, _L)
            val = plsc.load_gather(plane_v, [buf_v[sl]])
            buf_v[sl] = plsc.bitcast(val, jnp.int32)
        return j + 1

    pltpu.async_copy(tab_hbm.at[fp0], plane_v, semp)
    pltpu.sync_copy(idx_hbm.at[fp0 // _EMBED_DIM], buf_v)

    def plane_body(i, carry):
        fp = fp0 + i
        wait_plane(fp)
        lax.fori_loop(0, _N_UNITS, gather_inplace, 0)
        pltpu.async_copy(tab_hbm.at[fp + 1], plane_v, semp)
        pltpu.sync_copy(buf_v, out_hbm.at[fp])
        pltpu.sync_copy(idx_hbm.at[(fp + 1) // _EMBED_DIM], buf_v)
        return carry

    lax.fori_loop(0, _PLANES_PER_W - 1, plane_body, 0)
    fp_last = fp0 + _PLANES_PER_W - 1
    wait_plane(fp_last)
    lax.fori_loop(0, _N_UNITS, gather_inplace, 0)
    pltpu.sync_copy(buf_v, out_hbm.at[fp_last])


def kernel(inputs, tables):
    tab_t = jnp.transpose(tables, (0, 2, 1)).reshape(_PLANES, _VOCAB)
    idx_t = inputs.T.astype(jnp.int32)
    out_i = _lookup_kernel(tab_t, idx_t)
    out_t = lax.bitcast_convert_type(out_i, jnp.float32)
    return out_t.T.reshape(_BATCH, _PLANES)
